# baseline (speedup 1.0000x reference)
import functools

import jax
import jax.numpy as jnp
from jax import lax
from jax.experimental import pallas as pl
from jax.experimental.pallas import tpu as pltpu
from jax.experimental.pallas import tpu_sc as plsc

N_NODES = 10000
KNN_K = 32
_NP = 10240
_R = 128


def _knn_body(rows_ref, cols_ref, sqi_ref, sqj_ref, out_ref, *, n_valid, k):
    i = pl.program_id(0)
    rows = rows_ref[...]
    cols = cols_ref[...]
    npad = cols.shape[1]
    r = rows.shape[0]
    sqi = sqi_ref[...][:, :1]
    sqj = sqj_ref[...][:1, :]
    mm = jnp.dot(rows, cols, preferred_element_type=jnp.float32)
    s = (sqi + sqj) - 2.0 * mm
    col_iota = lax.broadcasted_iota(jnp.int32, (r, npad), 1)
    row_idx = i * r + lax.broadcasted_iota(jnp.int32, (r, npad), 0)
    s = s + jnp.where(col_iota == row_idx, jnp.float32(1e10), jnp.float32(0.0))
    s = jnp.where(col_iota >= n_valid, jnp.float32(jnp.inf), s)
    ncc = npad // 128
    sr = s.reshape(r, ncc, 128)
    lane3 = lax.broadcasted_iota(jnp.int32, (r, ncc, 128), 2)
    chunk_base = lax.broadcasted_iota(jnp.int32, (r, ncc), 1) * 128
    cvals, cgidx = [], []
    for _ in range(8):
        m = jnp.min(sr, axis=2, keepdims=True)
        li = jnp.min(jnp.where(sr <= m, lane3, 128), axis=2, keepdims=True)
        cvals.append(m[:, :, 0])
        cgidx.append(li[:, :, 0] + chunk_base)
        sr = jnp.where(lane3 == li, jnp.float32(jnp.inf), sr)
    cv = jnp.concatenate(cvals, axis=1)
    cg = jnp.concatenate(cgidx, axis=1)
    picks = []
    for _ in range(k):
        m = jnp.min(cv, axis=1, keepdims=True)
        gi = jnp.min(jnp.where(cv <= m, cg, npad), axis=1, keepdims=True)
        picks.append(gi)
        cv = jnp.where(cg == gi, jnp.float32(jnp.inf), cv)
    out_ref[...] = jnp.concatenate(picks, axis=1)


def _knn_pallas(x, k=KNN_K):
    n, f = x.shape
    fp = max(8, ((f + 7) // 8) * 8)
    xp = jnp.zeros((_NP, fp), jnp.float32).at[:n, :f].set(x)
    cols = xp.T
    sq = jnp.sum(x * x, axis=1)
    sqp = jnp.zeros((_NP,), jnp.float32).at[:n].set(sq)
    sqi_in = jnp.tile(sqp[:, None], (1, 8))
    sqj_in = jnp.tile(sqp[None, :], (8, 1))
    grid = (_NP // _R,)
    out = pl.pallas_call(
        functools.partial(_knn_body, n_valid=n, k=k),
        grid=grid,
        in_specs=[
            pl.BlockSpec((_R, fp), lambda i: (i, 0)),
            pl.BlockSpec((fp, _NP), lambda i: (0, 0)),
            pl.BlockSpec((_R, 8), lambda i: (i, 0)),
            pl.BlockSpec((8, _NP), lambda i: (0, 0)),
        ],
        out_specs=pl.BlockSpec((_R, k), lambda i: (i, 0)),
        out_shape=jax.ShapeDtypeStruct((_NP, k), jnp.int32),
    )(xp, cols, sqi_in, sqj_in)
    return out[:n]


def _mlp_apply(layers, h):
    for l in layers:
        h = h @ l["W"] + l["b"]
        h = jax.nn.relu(h)
        m = h.mean(0)
        v = h.var(0)
        h = (h - m) / jnp.sqrt(v + 1e-5) * l["g"] + l["be"]
    return h


def _dyn_edge_conv(layers, x, k):
    idx = _knn_pallas(x, k)
    n = x.shape[0]
    xi = jnp.broadcast_to(x[:, None, :], (n, k, x.shape[1]))
    xj = x[idx]
    h = jnp.concatenate([xi, xj - xi], axis=-1).reshape(n * k, -1)
    h = _mlp_apply(layers, h)
    return h.reshape(n, k, -1).max(axis=1)


_NW = 32
_NCH = 80
_NPAD_SC = 10112


def _make_sc_hop(d):
    mesh = plsc.VectorSubcoreMesh(core_axis_name="c", subcore_axis_name="s")
    rpt = _NPAD_SC // 16

    @functools.partial(
        pl.kernel, mesh=mesh,
        out_type=jax.ShapeDtypeStruct((2, _NPAD_SC, d), jnp.float32),
        scratch_types=[
            pltpu.VMEM((_NCH, 128), jnp.int32),
            pltpu.VMEM((_NCH, 128), jnp.int32),
            pltpu.VMEM((128, d), jnp.float32),
            pltpu.VMEM_SHARED((_NPAD_SC, d), jnp.float32),
            pltpu.SemaphoreType.DMA,
        ],
    )
    def hop(u_hbm, srcw_hbm, dstw_hbm, zeros_hbm, out_hbm,
            src_v, dst_v, rows_v, accum, sem):
        c = lax.axis_index("c")
        s = lax.axis_index("s")
        w = s * 2 + c
        pltpu.sync_copy(zeros_hbm.at[pl.ds(s * rpt, rpt)],
                        accum.at[pl.ds(s * rpt, rpt)])
        pltpu.sync_copy(srcw_hbm.at[w], src_v)
        pltpu.sync_copy(dstw_hbm.at[w], dst_v)
        plsc.subcore_barrier()

        def body(j, carry):
            pltpu.async_copy(u_hbm.at[src_v.at[j]], rows_v, sem).wait()
            pltpu.sync_copy(rows_v, accum.at[dst_v.at[j]], add=True)
            return carry

        lax.fori_loop(0, _NCH, body, 0)
        plsc.subcore_barrier()
        pltpu.sync_copy(accum.at[pl.ds(s * rpt, rpt)],
                        out_hbm.at[c, pl.ds(s * rpt, rpt)])

    return hop


_sc_hop_128 = _make_sc_hop(128)


def _sc_segment_sum(u_pad, srcw, dstw, zeros_pad, d):
    parts = _sc_hop_128(u_pad, srcw, dstw, zeros_pad)
    return parts[0] + parts[1]


def _pad_edges(src, dst, n):
    e_cap = _NW * _NCH * 128
    e = src.shape[0]
    srcp = jnp.full((e_cap,), n, jnp.int32).at[:e].set(src)
    dstp = jnp.full((e_cap,), n, jnp.int32).at[:e].set(dst)
    return (srcp.reshape(_NW, _NCH, 128), dstp.reshape(_NW, _NCH, 128))


def _tag_conv_sc(p, x, srcw, dstw, dis, n, hops=3):
    f = x.shape[1]
    d = 128
    zeros_pad = jnp.zeros((_NPAD_SC, d), jnp.float32)
    dis_col = dis[:, None]
    xs = [x]
    h = x
    for _ in range(hops):
        u = jnp.zeros((_NPAD_SC, d), jnp.float32).at[:n, :f].set(h * dis_col)
        s = _sc_segment_sum(u, srcw, dstw, zeros_pad, d)
        h = s[:n, :f] * dis_col
        xs.append(h)
    return jnp.concatenate(xs, axis=-1) @ p["W"] + p["b"]


def _degree_sc(srcw, dstw, n):
    ones = jnp.zeros((_NPAD_SC, 128), jnp.float32).at[:n, :1].set(1.0)
    zeros_pad = jnp.zeros((_NPAD_SC, 128), jnp.float32)
    s = _sc_segment_sum(ones, srcw, dstw, zeros_pad, 128)
    return s[:n, 0]


def kernel(pos, x, edge_index, params):
    src, dst = edge_index[0], edge_index[1]
    n = pos.shape[0]
    x1 = _dyn_edge_conv(params["conv1"], pos, KNN_K)
    x2 = _dyn_edge_conv(params["conv2"], x1, KNN_K)
    out_d = _mlp_apply(params["lin1"], jnp.concatenate([x1, x2], axis=-1))
    srcw, dstw = _pad_edges(src, dst, n)
    deg = _degree_sc(srcw, dstw, n)
    dis = jnp.where(deg > 0, 1.0 / jnp.sqrt(jnp.maximum(deg, 1.0)), 0.0)
    g1 = jax.nn.relu(_tag_conv_sc(params["tag1"], x, srcw, dstw, dis, n))
    g2 = jax.nn.relu(_tag_conv_sc(params["tag2"], g1, srcw, dstw, dis, n))
    out_g = _mlp_apply(params["lin_g1"], jnp.concatenate([g1, g2], axis=-1))
    h = jnp.concatenate([out_d, out_g], axis=-1)
    h = _mlp_apply(params["mix1"], h)
    h = _mlp_apply(params["mix2"], h)
    return h @ params["out"]["W"] + params["out"]["b"]


# speedup vs baseline: 1.9330x; 1.9330x over previous
import functools

import jax
import jax.numpy as jnp
from jax import lax
from jax.experimental import pallas as pl
from jax.experimental.pallas import tpu as pltpu
from jax.experimental.pallas import tpu_sc as plsc

N_NODES = 10000
KNN_K = 32
_NP = 10240
_R = 128


def _knn_body(xp_ref, colsb_ref, sqi_ref, sqj_ref, out_ref, *, n_valid, k):
    i = pl.program_id(0)
    xp = xp_ref[...]
    rows_t = colsb_ref[...]
    npad = xp.shape[0]
    r = rows_t.shape[1]
    sqj = sqi_ref[...][:, :1]
    sqi = sqj_ref[...][:1, :]
    mm = jnp.dot(xp, rows_t, preferred_element_type=jnp.float32)
    s = (sqi + sqj) - 2.0 * mm
    j_iota = lax.broadcasted_iota(jnp.int32, (npad, r), 0)
    i_glob = i * r + lax.broadcasted_iota(jnp.int32, (npad, r), 1)
    s = s + jnp.where(j_iota == i_glob, jnp.float32(1e10), jnp.float32(0.0))
    s = jnp.where(j_iota >= n_valid, jnp.float32(jnp.inf), s)
    ncc = npad // 128
    sr = s.reshape(ncc, 128, r)
    sub3 = lax.broadcasted_iota(jnp.int32, (ncc, 128, r), 1)
    chunk_base = lax.broadcasted_iota(jnp.int32, (ncc, 1, r), 0) * 128
    cvals, cgidx = [], []
    for _ in range(8):
        m = jnp.min(sr, axis=1, keepdims=True)
        li = jnp.min(jnp.where(sr <= m, sub3, 128), axis=1, keepdims=True)
        cvals.append(m)
        cgidx.append(li + chunk_base)
        sr = jnp.where(sub3 == li, jnp.float32(jnp.inf), sr)
    cv = jnp.concatenate(cvals, axis=1).reshape(ncc * 8, r)
    cg = jnp.concatenate(cgidx, axis=1).reshape(ncc * 8, r)
    picks = []
    for _ in range(k):
        m = jnp.min(cv, axis=0, keepdims=True)
        gi = jnp.min(jnp.where(cv <= m, cg, npad), axis=0, keepdims=True)
        picks.append(gi)
        cv = jnp.where(cg == gi, jnp.float32(jnp.inf), cv)
    out_ref[...] = jnp.concatenate(picks, axis=0)


def _knn_pallas(x, k=KNN_K):
    n, f = x.shape
    fp = max(8, ((f + 7) // 8) * 8)
    xp = jnp.zeros((_NP, fp), jnp.float32).at[:n, :f].set(x)
    cols = xp.T
    sq = jnp.sum(x * x, axis=1)
    sqp = jnp.zeros((_NP,), jnp.float32).at[:n].set(sq)
    sqi_in = jnp.tile(sqp[:, None], (1, 8))
    sqj_in = jnp.tile(sqp[None, :], (8, 1))
    grid = (_NP // _R,)
    out = pl.pallas_call(
        functools.partial(_knn_body, n_valid=n, k=k),
        grid=grid,
        in_specs=[
            pl.BlockSpec((_NP, fp), lambda i: (0, 0)),
            pl.BlockSpec((fp, _R), lambda i: (0, i)),
            pl.BlockSpec((_NP, 8), lambda i: (0, 0)),
            pl.BlockSpec((8, _R), lambda i: (0, i)),
        ],
        out_specs=pl.BlockSpec((k, _R), lambda i: (0, i)),
        out_shape=jax.ShapeDtypeStruct((k, _NP), jnp.int32),
    )(xp, cols, sqi_in, sqj_in)
    return out.T[:n]


def _mlp_apply(layers, h):
    for l in layers:
        h = h @ l["W"] + l["b"]
        h = jax.nn.relu(h)
        m = h.mean(0)
        v = h.var(0)
        h = (h - m) / jnp.sqrt(v + 1e-5) * l["g"] + l["be"]
    return h


def _dyn_edge_conv(layers, x, k):
    idx = _knn_pallas(x, k)
    n = x.shape[0]
    xi = jnp.broadcast_to(x[:, None, :], (n, k, x.shape[1]))
    xj = x[idx]
    h = jnp.concatenate([xi, xj - xi], axis=-1).reshape(n * k, -1)
    h = _mlp_apply(layers, h)
    return h.reshape(n, k, -1).max(axis=1)


_NW = 32
_NCH = 80
_NPAD_SC = 10112


def _make_sc_hop(d):
    mesh = plsc.VectorSubcoreMesh(core_axis_name="c", subcore_axis_name="s")
    rpt = _NPAD_SC // 16

    @functools.partial(
        pl.kernel, mesh=mesh,
        out_type=jax.ShapeDtypeStruct((2, _NPAD_SC, d), jnp.float32),
        scratch_types=[
            pltpu.VMEM((_NCH, 128), jnp.int32),
            pltpu.VMEM((_NCH, 128), jnp.int32),
            pltpu.VMEM((128, d), jnp.float32),
            pltpu.VMEM_SHARED((_NPAD_SC, d), jnp.float32),
            pltpu.SemaphoreType.DMA,
        ],
    )
    def hop(u_hbm, srcw_hbm, dstw_hbm, zeros_hbm, out_hbm,
            src_v, dst_v, rows_v, accum, sem):
        c = lax.axis_index("c")
        s = lax.axis_index("s")
        w = s * 2 + c
        pltpu.sync_copy(zeros_hbm.at[pl.ds(s * rpt, rpt)],
                        accum.at[pl.ds(s * rpt, rpt)])
        pltpu.sync_copy(srcw_hbm.at[w], src_v)
        pltpu.sync_copy(dstw_hbm.at[w], dst_v)
        plsc.subcore_barrier()

        def body(j, carry):
            pltpu.async_copy(u_hbm.at[src_v.at[j]], rows_v, sem).wait()
            pltpu.sync_copy(rows_v, accum.at[dst_v.at[j]], add=True)
            return carry

        lax.fori_loop(0, _NCH, body, 0)
        plsc.subcore_barrier()
        pltpu.sync_copy(accum.at[pl.ds(s * rpt, rpt)],
                        out_hbm.at[c, pl.ds(s * rpt, rpt)])

    return hop


_sc_hop_128 = _make_sc_hop(128)


def _sc_segment_sum(u_pad, srcw, dstw, zeros_pad, d):
    parts = _sc_hop_128(u_pad, srcw, dstw, zeros_pad)
    return parts[0] + parts[1]


def _pad_edges(src, dst, n):
    e_cap = _NW * _NCH * 128
    e = src.shape[0]
    srcp = jnp.full((e_cap,), n, jnp.int32).at[:e].set(src)
    dstp = jnp.full((e_cap,), n, jnp.int32).at[:e].set(dst)
    return (srcp.reshape(_NW, _NCH, 128), dstp.reshape(_NW, _NCH, 128))


def _tag_conv_sc(p, x, srcw, dstw, dis, n, hops=3):
    f = x.shape[1]
    d = 128
    zeros_pad = jnp.zeros((_NPAD_SC, d), jnp.float32)
    dis_col = dis[:, None]
    xs = [x]
    h = x
    for _ in range(hops):
        u = jnp.zeros((_NPAD_SC, d), jnp.float32).at[:n, :f].set(h * dis_col)
        s = _sc_segment_sum(u, srcw, dstw, zeros_pad, d)
        h = s[:n, :f] * dis_col
        xs.append(h)
    return jnp.concatenate(xs, axis=-1) @ p["W"] + p["b"]


def _degree_sc(srcw, dstw, n):
    ones = jnp.zeros((_NPAD_SC, 128), jnp.float32).at[:n, :1].set(1.0)
    zeros_pad = jnp.zeros((_NPAD_SC, 128), jnp.float32)
    s = _sc_segment_sum(ones, srcw, dstw, zeros_pad, 128)
    return s[:n, 0]


def kernel(pos, x, edge_index, params):
    src, dst = edge_index[0], edge_index[1]
    n = pos.shape[0]
    x1 = _dyn_edge_conv(params["conv1"], pos, KNN_K)
    x2 = _dyn_edge_conv(params["conv2"], x1, KNN_K)
    out_d = _mlp_apply(params["lin1"], jnp.concatenate([x1, x2], axis=-1))
    srcw, dstw = _pad_edges(src, dst, n)
    deg = _degree_sc(srcw, dstw, n)
    dis = jnp.where(deg > 0, 1.0 / jnp.sqrt(jnp.maximum(deg, 1.0)), 0.0)
    g1 = jax.nn.relu(_tag_conv_sc(params["tag1"], x, srcw, dstw, dis, n))
    g2 = jax.nn.relu(_tag_conv_sc(params["tag2"], g1, srcw, dstw, dis, n))
    out_g = _mlp_apply(params["lin_g1"], jnp.concatenate([g1, g2], axis=-1))
    h = jnp.concatenate([out_d, out_g], axis=-1)
    h = _mlp_apply(params["mix1"], h)
    h = _mlp_apply(params["mix2"], h)
    return h @ params["out"]["W"] + params["out"]["b"]


# speedup vs baseline: 1.9436x; 1.0055x over previous
import functools

import jax
import jax.numpy as jnp
from jax import lax
from jax.experimental import pallas as pl
from jax.experimental.pallas import tpu as pltpu
from jax.experimental.pallas import tpu_sc as plsc

N_NODES = 10000
KNN_K = 32
_NP = 10240
_R = 128


def _knn_body(xp_ref, colsb_ref, sqi_ref, sqj_ref, out_ref, *, n_valid, k):
    i = pl.program_id(0)
    xp = xp_ref[...]
    rows_t = colsb_ref[...]
    npad = xp.shape[0]
    r = rows_t.shape[1]
    sqj = sqi_ref[...][:, :1]
    sqi = sqj_ref[...][:1, :]
    mm = jnp.dot(xp, rows_t, preferred_element_type=jnp.float32)
    s = (sqi + sqj) - 2.0 * mm
    j_iota = lax.broadcasted_iota(jnp.int32, (npad, r), 0)
    i_glob = i * r + lax.broadcasted_iota(jnp.int32, (npad, r), 1)
    s = s + jnp.where(j_iota == i_glob, jnp.float32(1e10), jnp.float32(0.0))
    s = jnp.where(j_iota >= n_valid, jnp.float32(jnp.inf), s)
    ncc = npad // 128
    sr = s.reshape(ncc, 128, r)
    sub3 = lax.broadcasted_iota(jnp.int32, (ncc, 128, r), 1)
    chunk_base = lax.broadcasted_iota(jnp.int32, (ncc, 1, r), 0) * 128
    cvals, cgidx = [], []
    for _ in range(8):
        m = jnp.min(sr, axis=1, keepdims=True)
        li = jnp.min(jnp.where(sr <= m, sub3, 128), axis=1, keepdims=True)
        cvals.append(m)
        cgidx.append(li + chunk_base)
        sr = jnp.where(sub3 == li, jnp.float32(jnp.inf), sr)
    cv = jnp.concatenate(cvals, axis=1).reshape(ncc * 8, r)
    cg = jnp.concatenate(cgidx, axis=1).reshape(ncc * 8, r)
    picks = []
    for _ in range(k):
        m = jnp.min(cv, axis=0, keepdims=True)
        gi = jnp.min(jnp.where(cv <= m, cg, npad), axis=0, keepdims=True)
        picks.append(gi)
        cv = jnp.where(cg == gi, jnp.float32(jnp.inf), cv)
    out_ref[...] = jnp.concatenate(picks, axis=0)


def _knn_pallas(x, k=KNN_K):
    n, f = x.shape
    fp = max(8, ((f + 7) // 8) * 8)
    xp = jnp.zeros((_NP, fp), jnp.float32).at[:n, :f].set(x)
    cols = xp.T
    sq = jnp.sum(x * x, axis=1)
    sqp = jnp.zeros((_NP,), jnp.float32).at[:n].set(sq)
    sqi_in = jnp.tile(sqp[:, None], (1, 8))
    sqj_in = jnp.tile(sqp[None, :], (8, 1))
    grid = (_NP // _R,)
    out = pl.pallas_call(
        functools.partial(_knn_body, n_valid=n, k=k),
        grid=grid,
        in_specs=[
            pl.BlockSpec((_NP, fp), lambda i: (0, 0)),
            pl.BlockSpec((fp, _R), lambda i: (0, i)),
            pl.BlockSpec((_NP, 8), lambda i: (0, 0)),
            pl.BlockSpec((8, _R), lambda i: (0, i)),
        ],
        out_specs=pl.BlockSpec((k, _R), lambda i: (0, i)),
        out_shape=jax.ShapeDtypeStruct((k, _NP), jnp.int32),
    )(xp, cols, sqi_in, sqj_in)
    return out.T[:n]


def _mm_body(x_ref, w_ref, c_ref, m_ref, sd_ref, y_ref, s_ref, q_ref, *,
             relu, stats, maxgroup, normalize):
    x = x_ref[...]
    w = w_ref[...]
    c = c_ref[...][:1, :]
    if normalize:
        x = (x - m_ref[...][:1, :]) / sd_ref[...][:1, :]
    y = jnp.dot(x, w, preferred_element_type=jnp.float32) + c
    if relu:
        y = jnp.maximum(y, 0.0)
    if maxgroup:
        nb = y.shape[0] // 32
        y_ref[...] = jnp.max(y.reshape(nb, 32, y.shape[1]), axis=1)
    else:
        y_ref[...] = y
    if stats:
        s = jnp.broadcast_to(jnp.sum(y, axis=0, keepdims=True), s_ref.shape)
        q = jnp.broadcast_to(jnp.sum(y * y, axis=0, keepdims=True),
                             q_ref.shape)

        @pl.when(pl.program_id(0) == 0)
        def _():
            s_ref[...] = s
            q_ref[...] = q

        @pl.when(pl.program_id(0) != 0)
        def _():
            s_ref[...] += s
            q_ref[...] += q


def _mm_pallas(x, w, c=None, m=None, sd=None, *, relu=False, stats=False,
               maxgroup=False, blk=1000):
    rows, fin = x.shape
    fout = w.shape[1]
    assert rows % blk == 0
    grid = (rows // blk,)
    normalize = m is not None
    if c is None:
        c = jnp.zeros((fout,), jnp.float32)
    if m is None:
        m = jnp.zeros((fin,), jnp.float32)
        sd = jnp.ones((fin,), jnp.float32)
    c8 = jnp.tile(c[None, :], (8, 1))
    m8 = jnp.tile(m[None, :], (8, 1))
    sd8 = jnp.tile(sd[None, :], (8, 1))
    oblk = blk // 32 if maxgroup else blk
    orows = rows // 32 if maxgroup else rows
    outs = pl.pallas_call(
        functools.partial(_mm_body, relu=relu, stats=stats, maxgroup=maxgroup,
                          normalize=normalize),
        grid=grid,
        in_specs=[
            pl.BlockSpec((blk, fin), lambda i: (i, 0)),
            pl.BlockSpec((fin, fout), lambda i: (0, 0)),
            pl.BlockSpec((8, fout), lambda i: (0, 0)),
            pl.BlockSpec((8, fin), lambda i: (0, 0)),
            pl.BlockSpec((8, fin), lambda i: (0, 0)),
        ],
        out_specs=[
            pl.BlockSpec((oblk, fout), lambda i: (i, 0)),
            pl.BlockSpec((8, fout), lambda i: (0, 0)),
            pl.BlockSpec((8, fout), lambda i: (0, 0)),
        ],
        out_shape=[
            jax.ShapeDtypeStruct((orows, fout), jnp.float32),
            jax.ShapeDtypeStruct((8, fout), jnp.float32),
            jax.ShapeDtypeStruct((8, fout), jnp.float32),
        ],
    )(x, w, c8, m8, sd8)
    y, s, q = outs
    return y, s[0], q[0]


def _edge_layer_body(xi_ref, xj_ref, w_ref, y_ref, s_ref, q_ref, *, maxgroup):
    xi = xi_ref[...]
    xj = xj_ref[...]
    nb, f = xi.shape
    fo = w_ref.shape[1]
    xir = jnp.broadcast_to(xi[:, None, :], (nb, 32, f)).reshape(nb * 32, f)
    h = jnp.concatenate([xir, xj - xir], axis=1)
    y = jnp.maximum(jnp.dot(h, w_ref[...],
                            preferred_element_type=jnp.float32), 0.0)
    if maxgroup:
        y_ref[...] = jnp.max(y.reshape(nb, 32, fo), axis=1)
    else:
        y_ref[...] = y
    s = jnp.broadcast_to(jnp.sum(y, axis=0, keepdims=True), s_ref.shape)
    q = jnp.broadcast_to(jnp.sum(y * y, axis=0, keepdims=True), q_ref.shape)

    @pl.when(pl.program_id(0) == 0)
    def _():
        s_ref[...] = s
        q_ref[...] = q

    @pl.when(pl.program_id(0) != 0)
    def _():
        s_ref[...] += s
        q_ref[...] += q


def _edge_layer_pallas(xi, xjg, w, *, maxgroup, nblk=80):
    n, f = xi.shape
    fo = w.shape[1]
    grid = (n // nblk,)
    oblk = nblk if maxgroup else nblk * 32
    orows = n if maxgroup else n * 32
    outs = pl.pallas_call(
        functools.partial(_edge_layer_body, maxgroup=maxgroup),
        grid=grid,
        in_specs=[
            pl.BlockSpec((nblk, f), lambda i: (i, 0)),
            pl.BlockSpec((nblk * 32, f), lambda i: (i, 0)),
            pl.BlockSpec((2 * f, fo), lambda i: (0, 0)),
        ],
        out_specs=[
            pl.BlockSpec((oblk, fo), lambda i: (i, 0)),
            pl.BlockSpec((8, fo), lambda i: (0, 0)),
            pl.BlockSpec((8, fo), lambda i: (0, 0)),
        ],
        out_shape=[
            jax.ShapeDtypeStruct((orows, fo), jnp.float32),
            jax.ShapeDtypeStruct((8, fo), jnp.float32),
            jax.ShapeDtypeStruct((8, fo), jnp.float32),
        ],
    )(xi, xjg, w)
    y, s, q = outs
    return y, s[0], q[0]


def _bn_fold(w_next, m, v):
    sd = jnp.sqrt(v + 1e-5)
    return w_next / sd[:, None], -(m / sd) @ w_next


def _stats(s, q, count):
    m = s / count
    v = q / count - m * m
    return m, v


_NW = 32
_NCH = 80
_NPAD_SC = 10112


def _make_sc_hop(d):
    mesh = plsc.VectorSubcoreMesh(core_axis_name="c", subcore_axis_name="s")
    rpt = _NPAD_SC // 16

    @functools.partial(
        pl.kernel, mesh=mesh,
        out_type=jax.ShapeDtypeStruct((2, _NPAD_SC, d), jnp.float32),
        scratch_types=[
            pltpu.VMEM((_NCH, 128), jnp.int32),
            pltpu.VMEM((_NCH, 128), jnp.int32),
            pltpu.VMEM((128, d), jnp.float32),
            pltpu.VMEM_SHARED((_NPAD_SC, d), jnp.float32),
            pltpu.SemaphoreType.DMA,
        ],
    )
    def hop(u_hbm, srcw_hbm, dstw_hbm, zeros_hbm, out_hbm,
            src_v, dst_v, rows_v, accum, sem):
        c = lax.axis_index("c")
        s = lax.axis_index("s")
        w = s * 2 + c
        pltpu.sync_copy(zeros_hbm.at[pl.ds(s * rpt, rpt)],
                        accum.at[pl.ds(s * rpt, rpt)])
        pltpu.sync_copy(srcw_hbm.at[w], src_v)
        pltpu.sync_copy(dstw_hbm.at[w], dst_v)
        plsc.subcore_barrier()

        def body(j, carry):
            pltpu.async_copy(u_hbm.at[src_v.at[j]], rows_v, sem).wait()
            pltpu.sync_copy(rows_v, accum.at[dst_v.at[j]], add=True)
            return carry

        lax.fori_loop(0, _NCH, body, 0)
        plsc.subcore_barrier()
        pltpu.sync_copy(accum.at[pl.ds(s * rpt, rpt)],
                        out_hbm.at[c, pl.ds(s * rpt, rpt)])

    return hop


_sc_hop_cache = {}


def _sc_segment_sum(u_pad, srcw, dstw, zeros_pad, d):
    if 128 not in _sc_hop_cache:
        _sc_hop_cache[128] = _make_sc_hop(128)
    parts = _sc_hop_cache[128](u_pad, srcw, dstw, zeros_pad)
    return parts[0] + parts[1]


def _pad_edges(src, dst, n):
    e_cap = _NW * _NCH * 128
    e = src.shape[0]
    srcp = jnp.full((e_cap,), n, jnp.int32).at[:e].set(src)
    dstp = jnp.full((e_cap,), n, jnp.int32).at[:e].set(dst)
    return (srcp.reshape(_NW, _NCH, 128), dstp.reshape(_NW, _NCH, 128))


def _tag_hops_sc(x, srcw, dstw, dis, n, hops=3):
    f = x.shape[1]
    d = 128
    zeros_pad = jnp.zeros((_NPAD_SC, d), jnp.float32)
    dis_col = dis[:, None]
    xs = [x]
    h = x
    for _ in range(hops):
        u = jnp.zeros((_NPAD_SC, d), jnp.float32).at[:n, :f].set(h * dis_col)
        s = _sc_segment_sum(u, srcw, dstw, zeros_pad, d)
        h = s[:n, :f] * dis_col
        xs.append(h)
    return jnp.concatenate(xs, axis=-1)


def _degree_sc(srcw, dstw, n):
    ones = jnp.zeros((_NPAD_SC, 128), jnp.float32).at[:n, :1].set(1.0)
    zeros_pad = jnp.zeros((_NPAD_SC, 128), jnp.float32)
    s = _sc_segment_sum(ones, srcw, dstw, zeros_pad, 128)
    return s[:n, 0]


def kernel(pos, x, edge_index, params):
    src, dst = edge_index[0], edge_index[1]
    n = pos.shape[0]
    p = params
    e_cnt = n * KNN_K

    idx1 = _knn_pallas(pos, KNN_K)
    w1 = p["conv1"][0]["W"]
    w1p = jnp.zeros((16, 64), jnp.float32).at[0:3].set(w1[0:3]).at[8:11].set(w1[3:6])
    posp = jnp.zeros((n, 8), jnp.float32).at[:, :3].set(pos)
    xj1 = posp[idx1.reshape(-1)]
    e1, s1, q1 = _edge_layer_pallas(posp, xj1, w1p, maxgroup=False)
    m1, v1 = _stats(s1, q1, e_cnt)
    sd1 = jnp.sqrt(v1 + 1e-5)
    e2, s2, q2 = _mm_pallas(e1, p["conv1"][1]["W"], m=m1, sd=sd1,
                            relu=True, stats=True, blk=2560)
    m2, v2 = _stats(s2, q2, e_cnt)
    m1x, s3, q3 = _mm_pallas(e2, p["conv1"][2]["W"], m=m2,
                             sd=jnp.sqrt(v2 + 1e-5), relu=True, stats=True,
                             maxgroup=True, blk=2560)
    m3, v3 = _stats(s3, q3, e_cnt)
    x1 = (m1x - m3) / jnp.sqrt(v3 + 1e-5)

    idx2 = _knn_pallas(x1, KNN_K)
    xj2 = x1[idx2.reshape(-1)]
    m2x, s4, q4 = _edge_layer_pallas(x1, xj2, p["conv2"][0]["W"],
                                     maxgroup=True)
    m4, v4 = _stats(s4, q4, e_cnt)
    x2 = (m2x - m4) / jnp.sqrt(v4 + 1e-5)

    yd, s5, q5 = _mm_pallas(jnp.concatenate([x1, x2], axis=1),
                            p["lin1"][0]["W"], relu=True, stats=True)
    m5, v5 = _stats(s5, q5, n)

    srcw, dstw = _pad_edges(src, dst, n)
    deg = _degree_sc(srcw, dstw, n)
    dis = jnp.where(deg > 0, 1.0 / jnp.sqrt(jnp.maximum(deg, 1.0)), 0.0)
    cat_t1 = _tag_hops_sc(x, srcw, dstw, dis, n)
    g1 = _mm_pallas(cat_t1, p["tag1"]["W"], p["tag1"]["b"], relu=True)[0]
    cat_t2 = _tag_hops_sc(g1, srcw, dstw, dis, n)
    g2 = _mm_pallas(cat_t2, p["tag2"]["W"], p["tag2"]["b"], relu=True)[0]
    yg, s6, q6 = _mm_pallas(jnp.concatenate([g1, g2], axis=1),
                            p["lin_g1"][0]["W"], relu=True, stats=True)
    m6, v6 = _stats(s6, q6, n)

    cat3 = jnp.concatenate([yd, yg], axis=1)
    h1, s7, q7 = _mm_pallas(cat3, p["mix1"][0]["W"],
                            m=jnp.concatenate([m5, m6]),
                            sd=jnp.sqrt(jnp.concatenate([v5, v6]) + 1e-5),
                            relu=True, stats=True)
    m7, v7 = _stats(s7, q7, n)
    h2, s8, q8 = _mm_pallas(h1, p["mix2"][0]["W"], m=m7,
                            sd=jnp.sqrt(v7 + 1e-5), relu=True, stats=True)
    m8, v8 = _stats(s8, q8, n)
    return _mm_pallas(h2, p["out"]["W"], p["out"]["b"], m=m8,
                      sd=jnp.sqrt(v8 + 1e-5))[0]
